# Initial kernel scaffold; baseline (speedup 1.0000x reference)
import jax
import jax.numpy as jnp
from jax.experimental import pallas as pl


def kernel(x, edge_index, batch, W_in, b_in, W1, b1, W2, b2, fc1_W, fc1_b, fc2_W, fc2_b):
    raise NotImplementedError("write your pallas kernel here")


# speedup vs baseline: 10.0392x; 10.0392x over previous
import functools

import jax
import jax.numpy as jnp
from jax import lax
from jax.experimental import pallas as pl
from jax.experimental.pallas import tpu as pltpu
from jax.experimental.pallas import tpu_sc as plsc

N = 10000
EDG = 320000
D = 128
G = 16
NC, NS = 2, 16
NW = NC * NS
CHUNK = 128
NP = 10240
CPT = -(-EDG // (NW * CHUNK))
EP = CPT * NW * CHUNK
RPT = NP // NS
RBLK = 512
GRID = NP // RBLK


def _sc_mesh():
    return plsc.VectorSubcoreMesh(
        core_axis_name="c", subcore_axis_name="s", num_cores=NC, num_subcores=NS
    )


@functools.lru_cache(maxsize=None)
def _make_sc_deg():
    @functools.partial(
        pl.kernel,
        out_type=jax.ShapeDtypeStruct((NC, NP), jnp.float32),
        mesh=_sc_mesh(),
        scratch_types=[
            pltpu.VMEM_SHARED((NP,), jnp.float32),
            pltpu.VMEM((CHUNK,), jnp.int32),
            pltpu.VMEM((CHUNK,), jnp.float32),
        ],
    )
    def sc_deg(dst_hbm, zeros1_hbm, ones_hbm, deg_hbm, acc, idx_d, ones_v):
        c = lax.axis_index("c")
        s = lax.axis_index("s")
        wid = c * NS + s
        pltpu.sync_copy(
            zeros1_hbm.at[pl.ds(s * RPT, RPT)], acc.at[pl.ds(s * RPT, RPT)]
        )
        pltpu.sync_copy(ones_hbm, ones_v)
        plsc.subcore_barrier()

        def body(j, carry):
            base = (wid * CPT + j) * CHUNK
            pltpu.sync_copy(dst_hbm.at[pl.ds(base, CHUNK)], idx_d)
            pltpu.sync_copy(ones_v, acc.at[idx_d], add=True)
            return carry

        lax.fori_loop(0, CPT, body, 0)
        plsc.subcore_barrier()
        pltpu.sync_copy(
            acc.at[pl.ds(s * RPT, RPT)], deg_hbm.at[c, pl.ds(s * RPT, RPT)]
        )

    return sc_deg


@functools.lru_cache(maxsize=None)
def _make_sc_agg():
    @functools.partial(
        pl.kernel,
        out_type=jax.ShapeDtypeStruct((NC, NP, D), jnp.float32),
        mesh=_sc_mesh(),
        scratch_types=[
            pltpu.VMEM_SHARED((NP, D), jnp.float32),
            pltpu.VMEM((CHUNK,), jnp.int32),
            pltpu.VMEM((CHUNK,), jnp.int32),
            pltpu.VMEM((CHUNK, D), jnp.float32),
            pltpu.SemaphoreType.DMA,
        ],
    )
    def sc_agg(u_hbm, src_hbm, dst_hbm, zeros_hbm, agg_hbm, acc, idx_s, idx_d, rows, sem):
        c = lax.axis_index("c")
        s = lax.axis_index("s")
        wid = c * NS + s
        pltpu.sync_copy(
            zeros_hbm.at[pl.ds(s * RPT, RPT)], acc.at[pl.ds(s * RPT, RPT)]
        )
        plsc.subcore_barrier()

        def body(j, carry):
            base = (wid * CPT + j) * CHUNK
            pltpu.sync_copy(src_hbm.at[pl.ds(base, CHUNK)], idx_s)
            pltpu.sync_copy(dst_hbm.at[pl.ds(base, CHUNK)], idx_d)
            pltpu.async_copy(u_hbm.at[idx_s], rows, sem).wait()
            pltpu.sync_copy(rows, acc.at[idx_d], add=True)
            return carry

        lax.fori_loop(0, CPT, body, 0)
        plsc.subcore_barrier()
        pltpu.sync_copy(
            acc.at[pl.ds(s * RPT, RPT)], agg_hbm.at[c, pl.ds(s * RPT, RPT)]
        )

    return sc_agg


def _tca_body(x_ref, win_ref, bin_ref, w1_ref, dega_ref, degb_ref, t1_ref, u1_ref):
    t0 = jnp.dot(x_ref[...], win_ref[...], preferred_element_type=jnp.float32)
    t0 = t0 + bin_ref[...]
    t1 = jnp.dot(t0, w1_ref[...], preferred_element_type=jnp.float32)
    deg = dega_ref[...] + degb_ref[...] + 1.0
    dis = lax.rsqrt(deg)
    t1_ref[...] = t1
    u1_ref[...] = t1 * dis


def _tc_a(xp, W_in, b_in, W1, dega, degb):
    return pl.pallas_call(
        _tca_body,
        grid=(GRID,),
        in_specs=[
            pl.BlockSpec((RBLK, D), lambda i: (i, 0)),
            pl.BlockSpec((D, D), lambda i: (0, 0)),
            pl.BlockSpec((1, D), lambda i: (0, 0)),
            pl.BlockSpec((D, D), lambda i: (0, 0)),
            pl.BlockSpec((RBLK, 1), lambda i: (i, 0)),
            pl.BlockSpec((RBLK, 1), lambda i: (i, 0)),
        ],
        out_specs=[
            pl.BlockSpec((RBLK, D), lambda i: (i, 0)),
            pl.BlockSpec((RBLK, D), lambda i: (i, 0)),
        ],
        out_shape=[
            jax.ShapeDtypeStruct((NP, D), jnp.float32),
            jax.ShapeDtypeStruct((NP, D), jnp.float32),
        ],
    )(xp, W_in, b_in.reshape(1, D), W1, dega, degb)


def _tcb_body(a0_ref, a1_ref, t1_ref, dega_ref, degb_ref, b1_ref, w2_ref,
              t2_ref, u2_ref):
    deg = dega_ref[...] + degb_ref[...] + 1.0
    dis = lax.rsqrt(deg)
    invdeg = dis * dis
    h1 = (a0_ref[...] + a1_ref[...]) * dis + t1_ref[...] * invdeg + b1_ref[...]
    h1 = jnp.maximum(h1, 0.0)
    t2 = jnp.dot(h1, w2_ref[...], preferred_element_type=jnp.float32)
    t2_ref[...] = t2
    u2_ref[...] = t2 * dis


def _tc_b(a0, a1, t1, dega, degb, b1, W2):
    return pl.pallas_call(
        _tcb_body,
        grid=(GRID,),
        in_specs=[
            pl.BlockSpec((RBLK, D), lambda i: (i, 0)),
            pl.BlockSpec((RBLK, D), lambda i: (i, 0)),
            pl.BlockSpec((RBLK, D), lambda i: (i, 0)),
            pl.BlockSpec((RBLK, 1), lambda i: (i, 0)),
            pl.BlockSpec((RBLK, 1), lambda i: (i, 0)),
            pl.BlockSpec((1, D), lambda i: (0, 0)),
            pl.BlockSpec((D, D), lambda i: (0, 0)),
        ],
        out_specs=[
            pl.BlockSpec((RBLK, D), lambda i: (i, 0)),
            pl.BlockSpec((RBLK, D), lambda i: (i, 0)),
        ],
        out_shape=[
            jax.ShapeDtypeStruct((NP, D), jnp.float32),
            jax.ShapeDtypeStruct((NP, D), jnp.float32),
        ],
    )(a0, a1, t1, dega, degb, b1.reshape(1, D), W2)


def _tcc_body(a0_ref, a1_ref, t2_ref, dega_ref, degb_ref, b2_ref, batch_ref,
              fc1w_ref, fc1b_ref, fc2w_ref, fc2b_ref, out_ref,
              pooled_acc, counts_acc):
    i = pl.program_id(0)

    @pl.when(i == 0)
    def _init():
        pooled_acc[...] = jnp.zeros_like(pooled_acc)
        counts_acc[...] = jnp.zeros_like(counts_acc)

    deg = dega_ref[...] + degb_ref[...] + 1.0
    dis = lax.rsqrt(deg)
    invdeg = dis * dis
    h2 = (a0_ref[...] + a1_ref[...]) * dis + t2_ref[...] * invdeg + b2_ref[...]
    h2 = jnp.maximum(h2, 0.0)
    gids = lax.broadcasted_iota(jnp.int32, (RBLK, G), 1)
    oh = (batch_ref[...] == gids).astype(jnp.float32)
    pooled_acc[...] += lax.dot_general(
        oh, h2, (((0,), (0,)), ((), ())), preferred_element_type=jnp.float32
    )
    counts_acc[...] += lax.dot_general(
        oh, jnp.ones((RBLK, D), jnp.float32), (((0,), (0,)), ((), ())),
        preferred_element_type=jnp.float32,
    )

    @pl.when(i == GRID - 1)
    def _fin():
        cnt = jnp.maximum(counts_acc[...], 1.0)
        pooled = pooled_acc[...] / cnt
        z = jnp.dot(pooled, fc1w_ref[...], preferred_element_type=jnp.float32)
        z = jnp.maximum(z + fc1b_ref[...], 0.0)
        out = jnp.dot(z, fc2w_ref[...], preferred_element_type=jnp.float32)
        out_ref[...] = out + fc2b_ref[...]


def _tc_c(a0, a1, t2, dega, degb, b2, batchp, fc1_W, fc1_b, fc2_W, fc2_b):
    H2 = fc1_W.shape[1]
    C = fc2_W.shape[1]
    return pl.pallas_call(
        _tcc_body,
        grid=(GRID,),
        in_specs=[
            pl.BlockSpec((RBLK, D), lambda i: (i, 0)),
            pl.BlockSpec((RBLK, D), lambda i: (i, 0)),
            pl.BlockSpec((RBLK, D), lambda i: (i, 0)),
            pl.BlockSpec((RBLK, 1), lambda i: (i, 0)),
            pl.BlockSpec((RBLK, 1), lambda i: (i, 0)),
            pl.BlockSpec((1, D), lambda i: (0, 0)),
            pl.BlockSpec((RBLK, 1), lambda i: (i, 0)),
            pl.BlockSpec((D, H2), lambda i: (0, 0)),
            pl.BlockSpec((1, H2), lambda i: (0, 0)),
            pl.BlockSpec((H2, C), lambda i: (0, 0)),
            pl.BlockSpec((1, C), lambda i: (0, 0)),
        ],
        out_specs=pl.BlockSpec((G, C), lambda i: (0, 0)),
        out_shape=jax.ShapeDtypeStruct((G, C), jnp.float32),
        scratch_shapes=[
            pltpu.VMEM((G, D), jnp.float32),
            pltpu.VMEM((G, D), jnp.float32),
        ],
    )(a0, a1, t2, dega, degb, b2.reshape(1, D), batchp, fc1_W,
      fc1_b.reshape(1, H2), fc2_W, fc2_b.reshape(1, C))


def kernel(x, edge_index, batch, W_in, b_in, W1, b1, W2, b2, fc1_W, fc1_b,
           fc2_W, fc2_b):
    src = edge_index[0]
    dst = edge_index[1]
    pad_i = jnp.full((EP - EDG,), NP - 1, dtype=jnp.int32)
    srcp = jnp.concatenate([src, pad_i])
    dstp = jnp.concatenate([dst, pad_i])
    xp = jnp.zeros((NP, D), jnp.float32).at[:N].set(x)
    batchp = jnp.concatenate(
        [batch, jnp.full((NP - N,), G, dtype=jnp.int32)]
    ).reshape(NP, 1)
    zeros2 = jnp.zeros((NP, D), jnp.float32)
    zeros1 = jnp.zeros((NP,), jnp.float32)
    ones_c = jnp.ones((CHUNK,), jnp.float32)

    deg2 = _make_sc_deg()(dstp, zeros1, ones_c)
    dega = deg2[0].reshape(NP, 1)
    degb = deg2[1].reshape(NP, 1)

    t1, u1 = _tc_a(xp, W_in, b_in, W1, dega, degb)
    agg1 = _make_sc_agg()(u1, srcp, dstp, zeros2)
    t2, u2 = _tc_b(agg1[0], agg1[1], t1, dega, degb, b1, W2)
    agg2 = _make_sc_agg()(u2, srcp, dstp, zeros2)
    return _tc_c(agg2[0], agg2[1], t2, dega, degb, b2, batchp,
                 fc1_W, fc1_b, fc2_W, fc2_b)
